# Initial kernel scaffold; baseline (speedup 1.0000x reference)
import jax
import jax.numpy as jnp
from jax.experimental import pallas as pl


def kernel(idx, table):
    raise NotImplementedError("write your pallas kernel here")


# speedup vs baseline: 1.0260x; 1.0260x over previous
import functools

import jax
import jax.numpy as jnp
from jax import lax
from jax.experimental import pallas as pl
from jax.experimental.pallas import tpu as pltpu
from jax.experimental.pallas import tpu_sc as plsc

_NC = 2
_NS = 16
_NW = _NC * _NS


@functools.partial(jax.jit, static_argnames=("b", "t", "d"))
def _gather_sc(idx, table_p, b, t, d):
    dp = table_p.shape[1]
    b_per_w = b // _NW
    mesh = plsc.VectorSubcoreMesh(core_axis_name="c", subcore_axis_name="s")

    @functools.partial(
        pl.kernel,
        out_type=jax.ShapeDtypeStruct((b, t, d), jnp.float32),
        mesh=mesh,
        compiler_params=pltpu.CompilerParams(use_tc_tiling_on_sc=False),
        scratch_types=[
            pltpu.VMEM((b_per_w, t), jnp.int32),
            pltpu.VMEM((2, t, dp), jnp.float32),
            pltpu.SemaphoreType.DMA,
            pltpu.SemaphoreType.DMA,
        ],
    )
    def k(idx_hbm, table_hbm, out_hbm, idx_v, bufs, gsem, ssem):
        wid = lax.axis_index("s") * _NC + lax.axis_index("c")
        base = wid * b_per_w
        pltpu.sync_copy(idx_hbm.at[pl.ds(base, b_per_w)], idx_v)

        pltpu.make_async_copy(
            table_hbm.at[idx_v.at[0]], bufs.at[0], gsem
        ).start()

        @pl.loop(0, b_per_w)
        def _batch(j):
            s = lax.rem(j, 2)
            pltpu.make_async_copy(
                table_hbm.at[idx_v.at[j]], bufs.at[s], gsem
            ).wait()
            @pl.when(j + 1 < b_per_w)
            def _():
                pltpu.make_async_copy(
                    table_hbm.at[idx_v.at[j + 1]], bufs.at[1 - s], gsem
                ).start()
            pltpu.make_async_copy(
                bufs.at[s, :, pl.ds(0, d)], out_hbm.at[base + j], ssem
            ).start()
            pltpu.make_async_copy(
                bufs.at[s, :, pl.ds(0, d)], out_hbm.at[base + j], ssem
            ).wait()

    return k(idx, table_p)


def kernel(idx, table):
    b, t = idx.shape
    v, d = table.shape
    dp = (d + 127) // 128 * 128
    table_p = jnp.pad(table, ((0, 0), (0, dp - d)))
    return _gather_sc(idx.astype(jnp.int32), table_p, b, t, d)


# speedup vs baseline: 2.0743x; 2.0218x over previous
import functools

import jax
import jax.numpy as jnp
from jax import lax
from jax.experimental import pallas as pl
from jax.experimental.pallas import tpu as pltpu
from jax.experimental.pallas import tpu_sc as plsc

_NC = 2
_NS = 16
_NW = _NC * _NS


@functools.partial(jax.jit, static_argnames=("b", "t", "d"))
def _gather_sc(idx, table_p, b, t, d):
    dp = table_p.shape[1]
    b_per_w = b // _NW
    mesh = plsc.VectorSubcoreMesh(core_axis_name="c", subcore_axis_name="s")

    @functools.partial(
        pl.kernel,
        out_type=jax.ShapeDtypeStruct((b, t, dp), jnp.float32),
        mesh=mesh,
        scratch_types=[
            pltpu.VMEM((b_per_w, t), jnp.int32),
            pltpu.VMEM((2, t, dp), jnp.float32),
            pltpu.SemaphoreType.DMA,
            pltpu.SemaphoreType.DMA,
        ],
    )
    def k(idx_hbm, table_hbm, out_hbm, idx_v, bufs, gsem, ssem):
        wid = lax.axis_index("s") * _NC + lax.axis_index("c")
        base = wid * b_per_w
        pltpu.sync_copy(idx_hbm.at[pl.ds(base, b_per_w)], idx_v)

        pltpu.make_async_copy(
            table_hbm.at[idx_v.at[0]], bufs.at[0], gsem
        ).start()

        @pl.loop(0, b_per_w)
        def _batch(j):
            s = lax.rem(j, 2)
            pltpu.make_async_copy(
                table_hbm.at[idx_v.at[j]], bufs.at[s], gsem
            ).wait()
            @pl.when(j + 1 < b_per_w)
            def _():
                pltpu.make_async_copy(
                    table_hbm.at[idx_v.at[j + 1]], bufs.at[1 - s], gsem
                ).start()
            pltpu.make_async_copy(
                bufs.at[s], out_hbm.at[base + j], ssem
            ).start()
            pltpu.make_async_copy(
                bufs.at[s], out_hbm.at[base + j], ssem
            ).wait()

    return k(idx, table_p)[:, :, :d]


def kernel(idx, table):
    b, t = idx.shape
    v, d = table.shape
    dp = (d + 127) // 128 * 128
    table_p = jnp.pad(table, ((0, 0), (0, dp - d)))
    return _gather_sc(idx.astype(jnp.int32), table_p, b, t, d)
